# baseline (speedup 1.0000x reference)
import functools

import jax
import jax.numpy as jnp
from jax.experimental import pallas as pl
from jax.experimental.pallas import tpu as pltpu

_C = 128
_NEWTON = 6


def _dpfp1(z):
    xp = jnp.concatenate([jax.nn.relu(z), jax.nn.relu(-z)], axis=-1)
    rolled = jnp.concatenate([xp[:, -1:], xp[:, :-1]], axis=-1)
    y = xp * rolled
    return y / (jnp.sum(y, axis=-1, keepdims=True) + 1e-6)


def _fwa_body(H, DK, B, x_ref, x2_ref, wq_ref, wk_ref, wv_ref, wg_ref,
              wo_ref, bo_ref, o_ref, wstate, tscr, qkscr, qscr, kscr,
              kbscr, bvscr):
    C = _C
    PHI = 2 * DK
    D = wo_ref.shape[0]
    f32 = jnp.float32
    bf16 = jnp.bfloat16
    bhs = [(b, h) for b in range(B) for h in range(H)]
    c = pl.program_id(0)

    ri = jax.lax.broadcasted_iota(jnp.int32, (C, C), 0)
    ci = jax.lax.broadcasted_iota(jnp.int32, (C, C), 1)
    eye = (ri == ci).astype(f32)
    strict = (ri > ci).astype(f32)
    incl = (ri >= ci).astype(f32)

    wqh = wq_ref[...].astype(bf16)
    wkh = wk_ref[...].astype(bf16)
    wvh = wv_ref[...].astype(bf16)
    wgh = wg_ref[...].astype(bf16)

    def _prep(xfull_f32):
        qa, ka, va, ba = [], [], [], []
        for b in range(B):
            xb = xfull_f32[:, b * D:(b + 1) * D].astype(bf16)
            qa.append(jnp.dot(xb, wqh, preferred_element_type=f32))
            ka.append(jnp.dot(xb, wkh, preferred_element_type=f32))
            va.append(jnp.dot(xb, wvh, preferred_element_type=f32))
            ba.append(jax.nn.sigmoid(
                jnp.dot(xb, wgh, preferred_element_type=f32)))

        qs = [_dpfp1(qa[b][:, h * DK:(h + 1) * DK]).astype(bf16)
              for b, h in bhs]
        ks = [_dpfp1(ka[b][:, h * DK:(h + 1) * DK]) for b, h in bhs]
        betas = [ba[b][:, h:h + 1] for b, h in bhs]
        khs = [k.astype(bf16) for k in ks]
        kbs = [(ks[i] * betas[i]).astype(bf16) for i in range(B * H)]
        bvs = [(betas[i] * va[b][:, h * DK:(h + 1) * DK]).astype(bf16)
               for i, (b, h) in enumerate(bhs)]

        a_s = [strict * jax.lax.dot_general(
            kbs[i], khs[i], (((1,), (1,)), ((), ())),
            preferred_element_type=f32) for i in range(B * H)]
        ts = [(eye - a).astype(bf16) for a in a_s]
        ls = [(eye + a).astype(bf16) for a in a_s]
        for _ in range(_NEWTON):
            inners = [jnp.dot(ls[i], ts[i], preferred_element_type=f32)
                      for i in range(B * H)]
            ts = [jnp.dot(ts[i], (2.0 * eye - inners[i]).astype(bf16),
                          preferred_element_type=f32).astype(bf16)
                  for i in range(B * H)]
        qks = [(incl * jax.lax.dot_general(
            qs[i], khs[i], (((1,), (1,)), ((), ())),
            preferred_element_type=f32)).astype(bf16) for i in range(B * H)]

        for i in range(B * H):
            tscr[i] = ts[i]
            qkscr[i] = qks[i]
            qscr[:, i * PHI:(i + 1) * PHI] = qs[i]
            kscr[:, i * PHI:(i + 1) * PHI] = khs[i]
            kbscr[:, i * PHI:(i + 1) * PHI] = kbs[i]
            bvscr[:, i * DK:(i + 1) * DK] = bvs[i]

    @pl.when(c == 0)
    def _():
        wstate[...] = jnp.zeros_like(wstate)
        _prep(x_ref[...])

    woh = wo_ref[...].astype(bf16)
    wts = [wstate[i] for i in range(B * H)]
    wths = [w.astype(bf16) for w in wts]
    b_rhss = [(bvscr[:, i * DK:(i + 1) * DK]
               - jnp.dot(kbscr[:, i * PHI:(i + 1) * PHI], wths[i],
                         preferred_element_type=f32)).astype(bf16)
              for i in range(B * H)]
    us = [jnp.dot(tscr[i], b_rhss[i], preferred_element_type=f32)
          for i in range(B * H)]
    uhs = [u.astype(bf16) for u in us]
    o_hs = [jnp.dot(qscr[:, i * PHI:(i + 1) * PHI], wths[i],
                    preferred_element_type=f32)
            + jnp.dot(qkscr[i], uhs[i], preferred_element_type=f32)
            for i in range(B * H)]
    for i in range(B * H):
        wstate[i] = wts[i] + jax.lax.dot_general(
            kscr[:, i * PHI:(i + 1) * PHI], uhs[i],
            (((0,), (0,)), ((), ())), preferred_element_type=f32)

    for b in range(B):
        o_full = jnp.concatenate(
            o_hs[b * H:(b + 1) * H], axis=-1).astype(bf16)
        o_ref[:, b * D:(b + 1) * D] = (
            jnp.dot(o_full, woh, preferred_element_type=f32) + bo_ref[0, :])

    _prep(x2_ref[...])


def kernel(x, Wq, Wk, Wv, Wg, Wo, bo):
    S, B, D = x.shape
    H = Wg.shape[1]
    DK = Wq.shape[1] // H
    PHI = 2 * DK
    C = _C
    NC = S // C

    x2d = x.reshape(S, B * D)
    body = functools.partial(_fwa_body, H, DK, B)
    out = pl.pallas_call(
        body,
        grid=(NC,),
        in_specs=[
            pl.BlockSpec((C, B * D), lambda c: (c, 0)),
            pl.BlockSpec((C, B * D), lambda c: (jnp.minimum(c + 1, NC - 1),
                                                0)),
            pl.BlockSpec((D, H * DK), lambda c: (0, 0)),
            pl.BlockSpec((D, H * DK), lambda c: (0, 0)),
            pl.BlockSpec((D, H * DK), lambda c: (0, 0)),
            pl.BlockSpec((D, H), lambda c: (0, 0)),
            pl.BlockSpec((D, D), lambda c: (0, 0)),
            pl.BlockSpec((1, D), lambda c: (0, 0)),
        ],
        out_specs=pl.BlockSpec((C, B * D), lambda c: (c, 0)),
        out_shape=jax.ShapeDtypeStruct((S, B * D), x.dtype),
        scratch_shapes=[
            pltpu.VMEM((B * H, PHI, DK), jnp.float32),
            pltpu.VMEM((B * H, C, C), jnp.bfloat16),
            pltpu.VMEM((B * H, C, C), jnp.bfloat16),
            pltpu.VMEM((C, B * H * PHI), jnp.bfloat16),
            pltpu.VMEM((C, B * H * PHI), jnp.bfloat16),
            pltpu.VMEM((C, B * H * PHI), jnp.bfloat16),
            pltpu.VMEM((C, B * H * DK), jnp.bfloat16),
        ],
        compiler_params=pltpu.CompilerParams(
            dimension_semantics=("arbitrary",)),
    )(x2d, x2d, Wq, Wk, Wv, Wg, Wo, bo.reshape(1, D))
    return out.reshape(S, B, D)


# speedup vs baseline: 1.2688x; 1.2688x over previous
import functools

import jax
import jax.numpy as jnp
from jax.experimental import pallas as pl
from jax.experimental.pallas import tpu as pltpu

_C = 128
_NEWTON = 6


def _dpfp1(z):
    xp = jnp.concatenate([jax.nn.relu(z), jax.nn.relu(-z)], axis=-1)
    rolled = jnp.concatenate([xp[:, -1:], xp[:, :-1]], axis=-1)
    y = xp * rolled
    return y / (jnp.sum(y, axis=-1, keepdims=True) + 1e-6)


def _fwa_body(H, DK, x_ref, x2_ref, wq_ref, wk_ref, wv_ref, wg_ref, wo_ref,
              bo_ref, o_ref, wstate, tscr, qkscr, qscr, kscr, kbscr, bvscr):
    C = _C
    PHI = 2 * DK
    f32 = jnp.float32
    bf16 = jnp.bfloat16
    hs = range(H)
    c = pl.program_id(1)

    ri = jax.lax.broadcasted_iota(jnp.int32, (C, C), 0)
    ci = jax.lax.broadcasted_iota(jnp.int32, (C, C), 1)
    eye = (ri == ci).astype(f32)
    strict = (ri > ci).astype(f32)
    incl = (ri >= ci).astype(f32)

    def _prep(xb_f32):
        xb = xb_f32.astype(bf16)
        q_all = jnp.dot(xb, wq_ref[...].astype(bf16),
                        preferred_element_type=f32)
        k_all = jnp.dot(xb, wk_ref[...].astype(bf16),
                        preferred_element_type=f32)
        v_all = jnp.dot(xb, wv_ref[...].astype(bf16),
                        preferred_element_type=f32)
        beta_all = jax.nn.sigmoid(
            jnp.dot(xb, wg_ref[...].astype(bf16),
                    preferred_element_type=f32))

        qs = [_dpfp1(q_all[:, h * DK:(h + 1) * DK]).astype(bf16) for h in hs]
        ks = [_dpfp1(k_all[:, h * DK:(h + 1) * DK]) for h in hs]
        betas = [beta_all[:, h:h + 1] for h in hs]
        khs = [ks[h].astype(bf16) for h in hs]
        kbs = [(ks[h] * betas[h]).astype(bf16) for h in hs]
        bvs = [(betas[h] * v_all[:, h * DK:(h + 1) * DK]).astype(bf16)
               for h in hs]

        a_s = [strict * jax.lax.dot_general(
            kbs[h], khs[h], (((1,), (1,)), ((), ())),
            preferred_element_type=f32) for h in hs]
        ts = [(eye - a_s[h]).astype(bf16) for h in hs]
        ls = [(eye + a_s[h]).astype(bf16) for h in hs]
        for _ in range(_NEWTON):
            inners = [jnp.dot(ls[h], ts[h], preferred_element_type=f32)
                      for h in hs]
            ts = [jnp.dot(ts[h], (2.0 * eye - inners[h]).astype(bf16),
                          preferred_element_type=f32).astype(bf16)
                  for h in hs]
        qks = [(incl * jax.lax.dot_general(
            qs[h], khs[h], (((1,), (1,)), ((), ())),
            preferred_element_type=f32)).astype(bf16) for h in hs]

        for h in hs:
            tscr[h] = ts[h]
            qkscr[h] = qks[h]
            qscr[:, h * PHI:(h + 1) * PHI] = qs[h]
            kscr[:, h * PHI:(h + 1) * PHI] = khs[h]
            kbscr[:, h * PHI:(h + 1) * PHI] = kbs[h]
            bvscr[:, h * DK:(h + 1) * DK] = bvs[h]

    @pl.when(c == 0)
    def _():
        wstate[...] = jnp.zeros_like(wstate)
        _prep(x_ref[...])

    wts = [wstate[h] for h in hs]
    wths = [wts[h].astype(bf16) for h in hs]
    b_rhss = [(bvscr[:, h * DK:(h + 1) * DK]
               - jnp.dot(kbscr[:, h * PHI:(h + 1) * PHI], wths[h],
                         preferred_element_type=f32)).astype(bf16)
              for h in hs]
    us = [jnp.dot(tscr[h], b_rhss[h], preferred_element_type=f32)
          for h in hs]
    uhs = [us[h].astype(bf16) for h in hs]
    o_hs = [jnp.dot(qscr[:, h * PHI:(h + 1) * PHI], wths[h],
                    preferred_element_type=f32)
            + jnp.dot(qkscr[h], uhs[h], preferred_element_type=f32)
            for h in hs]
    for h in hs:
        wstate[h] = wts[h] + jax.lax.dot_general(
            kscr[:, h * PHI:(h + 1) * PHI], uhs[h],
            (((0,), (0,)), ((), ())), preferred_element_type=f32)

    o_full = jnp.concatenate(o_hs, axis=-1).astype(bf16)
    o_ref[...] = (jnp.dot(o_full, wo_ref[...].astype(bf16),
                          preferred_element_type=f32)
                  + bo_ref[0, :])

    _prep(x2_ref[...])


def kernel(x, Wq, Wk, Wv, Wg, Wo, bo):
    S, B, D = x.shape
    H = Wg.shape[1]
    DK = Wq.shape[1] // H
    PHI = 2 * DK
    C = _C
    NC = S // C

    x2d = x.reshape(S, B * D)
    body = functools.partial(_fwa_body, H, DK)
    out = pl.pallas_call(
        body,
        grid=(B, NC),
        in_specs=[
            pl.BlockSpec((C, D), lambda b, c: (c, b)),
            pl.BlockSpec((C, D),
                         lambda b, c: (jnp.minimum(c + 1, NC - 1), b)),
            pl.BlockSpec((D, H * DK), lambda b, c: (0, 0)),
            pl.BlockSpec((D, H * DK), lambda b, c: (0, 0)),
            pl.BlockSpec((D, H * DK), lambda b, c: (0, 0)),
            pl.BlockSpec((D, H), lambda b, c: (0, 0)),
            pl.BlockSpec((D, D), lambda b, c: (0, 0)),
            pl.BlockSpec((1, D), lambda b, c: (0, 0)),
        ],
        out_specs=pl.BlockSpec((C, D), lambda b, c: (c, b)),
        out_shape=jax.ShapeDtypeStruct((S, B * D), x.dtype),
        scratch_shapes=[
            pltpu.VMEM((H, PHI, DK), jnp.float32),
            pltpu.VMEM((H, C, C), jnp.bfloat16),
            pltpu.VMEM((H, C, C), jnp.bfloat16),
            pltpu.VMEM((C, H * PHI), jnp.bfloat16),
            pltpu.VMEM((C, H * PHI), jnp.bfloat16),
            pltpu.VMEM((C, H * PHI), jnp.bfloat16),
            pltpu.VMEM((C, H * DK), jnp.bfloat16),
        ],
        compiler_params=pltpu.CompilerParams(
            dimension_semantics=("parallel", "arbitrary")),
    )(x2d, x2d, Wq, Wk, Wv, Wg, Wo, bo.reshape(1, D))
    return out.reshape(S, B, D)


# speedup vs baseline: 1.6364x; 1.2898x over previous
import functools

import jax
import jax.numpy as jnp
from jax.experimental import pallas as pl
from jax.experimental.pallas import tpu as pltpu

_C = 128
_G = 2
_NEWTON = 6


def _dpfp1(z):
    xp = jnp.concatenate([jax.nn.relu(z), jax.nn.relu(-z)], axis=-1)
    rolled = jnp.concatenate([xp[:, -1:], xp[:, :-1]], axis=-1)
    y = xp * rolled
    return y / (jnp.sum(y, axis=-1, keepdims=True) + 1e-6)


def _fwa_body(H, DK, G, x_ref, x2_ref, wq_ref, wk_ref, wv_ref, wg_ref,
              wo_ref, bo_ref, o_ref, wstate, tscr, qkscr, qscr, kscr,
              kbscr, bvscr):
    C = _C
    PHI = 2 * DK
    D = wo_ref.shape[0]
    f32 = jnp.float32
    bf16 = jnp.bfloat16
    hs = [(g, h) for g in range(G) for h in range(H)]
    NH = G * H
    c = pl.program_id(1)

    ri = jax.lax.broadcasted_iota(jnp.int32, (C, C), 0)
    ci = jax.lax.broadcasted_iota(jnp.int32, (C, C), 1)
    eye = (ri == ci).astype(f32)
    strict = (ri > ci).astype(f32)
    incl = (ri >= ci).astype(f32)

    wqh = wq_ref[...].astype(bf16)
    wkh = wk_ref[...].astype(bf16)
    wvh = wv_ref[...].astype(bf16)
    wgh = wg_ref[...].astype(bf16)

    def _prep(xfull_f32):
        q_a, k_a, v_a, b_a = [], [], [], []
        for g in range(G):
            xb = xfull_f32[:, g * D:(g + 1) * D].astype(bf16)
            q_a.append(jnp.dot(xb, wqh, preferred_element_type=f32))
            k_a.append(jnp.dot(xb, wkh, preferred_element_type=f32))
            v_a.append(jnp.dot(xb, wvh, preferred_element_type=f32))
            b_a.append(jax.nn.sigmoid(
                jnp.dot(xb, wgh, preferred_element_type=f32)))

        qs = [_dpfp1(q_a[g][:, h * DK:(h + 1) * DK]).astype(bf16)
              for g, h in hs]
        ks = [_dpfp1(k_a[g][:, h * DK:(h + 1) * DK]) for g, h in hs]
        betas = [b_a[g][:, h:h + 1] for g, h in hs]
        khs = [k.astype(bf16) for k in ks]
        kbs = [(ks[i] * betas[i]).astype(bf16) for i in range(NH)]
        bvs = [(betas[i] * v_a[g][:, h * DK:(h + 1) * DK]).astype(bf16)
               for i, (g, h) in enumerate(hs)]

        a_s = [strict * jax.lax.dot_general(
            kbs[i], khs[i], (((1,), (1,)), ((), ())),
            preferred_element_type=f32) for i in range(NH)]
        ts = [(eye - a).astype(bf16) for a in a_s]
        ls = [(eye + a).astype(bf16) for a in a_s]
        for _ in range(_NEWTON):
            inners = [jnp.dot(ls[i], ts[i], preferred_element_type=f32)
                      for i in range(NH)]
            ts = [jnp.dot(ts[i], (2.0 * eye - inners[i]).astype(bf16),
                          preferred_element_type=f32).astype(bf16)
                  for i in range(NH)]
        qks = [(incl * jax.lax.dot_general(
            qs[i], khs[i], (((1,), (1,)), ((), ())),
            preferred_element_type=f32)).astype(bf16) for i in range(NH)]

        for i in range(NH):
            tscr[i] = ts[i]
            qkscr[i] = qks[i]
            qscr[:, i * PHI:(i + 1) * PHI] = qs[i]
            kscr[:, i * PHI:(i + 1) * PHI] = khs[i]
            kbscr[:, i * PHI:(i + 1) * PHI] = kbs[i]
            bvscr[:, i * DK:(i + 1) * DK] = bvs[i]

    @pl.when(c == 0)
    def _():
        wstate[...] = jnp.zeros_like(wstate)
        _prep(x_ref[...])

    woh = wo_ref[...].astype(bf16)
    wts = [wstate[i] for i in range(NH)]
    wths = [w.astype(bf16) for w in wts]
    b_rhss = [(bvscr[:, i * DK:(i + 1) * DK]
               - jnp.dot(kbscr[:, i * PHI:(i + 1) * PHI], wths[i],
                         preferred_element_type=f32)).astype(bf16)
              for i in range(NH)]
    us = [jnp.dot(tscr[i], b_rhss[i], preferred_element_type=f32)
          for i in range(NH)]
    uhs = [u.astype(bf16) for u in us]
    o_hs = [jnp.dot(qscr[:, i * PHI:(i + 1) * PHI], wths[i],
                    preferred_element_type=f32)
            + jnp.dot(qkscr[i], uhs[i], preferred_element_type=f32)
            for i in range(NH)]
    for i in range(NH):
        wstate[i] = wts[i] + jax.lax.dot_general(
            kscr[:, i * PHI:(i + 1) * PHI], uhs[i],
            (((0,), (0,)), ((), ())), preferred_element_type=f32)

    for g in range(G):
        o_full = jnp.concatenate(
            o_hs[g * H:(g + 1) * H], axis=-1).astype(bf16)
        o_ref[:, g * D:(g + 1) * D] = (
            jnp.dot(o_full, woh, preferred_element_type=f32) + bo_ref[0, :])

    _prep(x2_ref[...])


def kernel(x, Wq, Wk, Wv, Wg, Wo, bo):
    S, B, D = x.shape
    H = Wg.shape[1]
    DK = Wq.shape[1] // H
    PHI = 2 * DK
    C = _C
    NC = S // C

    G = _G
    x2d = x.reshape(S, B * D)
    body = functools.partial(_fwa_body, H, DK, G)
    out = pl.pallas_call(
        body,
        grid=(B // G, NC),
        in_specs=[
            pl.BlockSpec((C, G * D), lambda b, c: (c, b)),
            pl.BlockSpec((C, G * D),
                         lambda b, c: (jnp.minimum(c + 1, NC - 1), b)),
            pl.BlockSpec((D, H * DK), lambda b, c: (0, 0)),
            pl.BlockSpec((D, H * DK), lambda b, c: (0, 0)),
            pl.BlockSpec((D, H * DK), lambda b, c: (0, 0)),
            pl.BlockSpec((D, H), lambda b, c: (0, 0)),
            pl.BlockSpec((D, D), lambda b, c: (0, 0)),
            pl.BlockSpec((1, D), lambda b, c: (0, 0)),
        ],
        out_specs=pl.BlockSpec((C, G * D), lambda b, c: (c, b)),
        out_shape=jax.ShapeDtypeStruct((S, B * D), x.dtype),
        scratch_shapes=[
            pltpu.VMEM((G * H, PHI, DK), jnp.float32),
            pltpu.VMEM((G * H, C, C), jnp.bfloat16),
            pltpu.VMEM((G * H, C, C), jnp.bfloat16),
            pltpu.VMEM((C, G * H * PHI), jnp.bfloat16),
            pltpu.VMEM((C, G * H * PHI), jnp.bfloat16),
            pltpu.VMEM((C, G * H * PHI), jnp.bfloat16),
            pltpu.VMEM((C, G * H * DK), jnp.bfloat16),
        ],
        compiler_params=pltpu.CompilerParams(
            dimension_semantics=("parallel", "arbitrary")),
    )(x2d, x2d, Wq, Wk, Wv, Wg, Wo, bo.reshape(1, D))
    return out.reshape(S, B, D)


# speedup vs baseline: 1.6757x; 1.0240x over previous
import functools

import jax
import jax.numpy as jnp
from jax.experimental import pallas as pl
from jax.experimental.pallas import tpu as pltpu

_C = 128
_G = 4
_NEWTON = 6


def _dpfp1(z):
    xp = jnp.concatenate([jax.nn.relu(z), jax.nn.relu(-z)], axis=-1)
    rolled = jnp.concatenate([xp[:, -1:], xp[:, :-1]], axis=-1)
    y = xp * rolled
    return y / (jnp.sum(y, axis=-1, keepdims=True) + 1e-6)


def _fwa_body(H, DK, G, x_ref, x2_ref, wq_ref, wk_ref, wv_ref, wg_ref,
              wo_ref, bo_ref, o_ref, wstate, tscr, qkscr, qscr, kscr,
              kbscr, bvscr):
    C = _C
    PHI = 2 * DK
    D = wo_ref.shape[0]
    f32 = jnp.float32
    bf16 = jnp.bfloat16
    hs = [(g, h) for g in range(G) for h in range(H)]
    NH = G * H
    c = pl.program_id(1)

    ri = jax.lax.broadcasted_iota(jnp.int32, (C, C), 0)
    ci = jax.lax.broadcasted_iota(jnp.int32, (C, C), 1)
    eye = (ri == ci).astype(f32)
    strict = (ri > ci).astype(f32)
    incl = (ri >= ci).astype(f32)

    wqh = wq_ref[...].astype(bf16)
    wkh = wk_ref[...].astype(bf16)
    wvh = wv_ref[...].astype(bf16)
    wgh = wg_ref[...].astype(bf16)

    def _prep(xfull_f32):
        q_a, k_a, v_a, b_a = [], [], [], []
        for g in range(G):
            xb = xfull_f32[:, g * D:(g + 1) * D].astype(bf16)
            q_a.append(jnp.dot(xb, wqh, preferred_element_type=f32))
            k_a.append(jnp.dot(xb, wkh, preferred_element_type=f32))
            v_a.append(jnp.dot(xb, wvh, preferred_element_type=f32))
            b_a.append(jax.nn.sigmoid(
                jnp.dot(xb, wgh, preferred_element_type=f32)))

        qs = [_dpfp1(q_a[g][:, h * DK:(h + 1) * DK]).astype(bf16)
              for g, h in hs]
        ks = [_dpfp1(k_a[g][:, h * DK:(h + 1) * DK]) for g, h in hs]
        betas = [b_a[g][:, h:h + 1] for g, h in hs]
        khs = [k.astype(bf16) for k in ks]
        kbs = [(ks[i] * betas[i]).astype(bf16) for i in range(NH)]
        bvs = [(betas[i] * v_a[g][:, h * DK:(h + 1) * DK]).astype(bf16)
               for i, (g, h) in enumerate(hs)]

        a_s = [strict * jax.lax.dot_general(
            kbs[i], khs[i], (((1,), (1,)), ((), ())),
            preferred_element_type=f32) for i in range(NH)]
        ts = [(eye - a).astype(bf16) for a in a_s]
        ls = [(eye + a).astype(bf16) for a in a_s]
        for _ in range(_NEWTON):
            inners = [jnp.dot(ls[i], ts[i], preferred_element_type=f32)
                      for i in range(NH)]
            ts = [jnp.dot(ts[i], (2.0 * eye - inners[i]).astype(bf16),
                          preferred_element_type=f32).astype(bf16)
                  for i in range(NH)]
        qks = [(incl * jax.lax.dot_general(
            qs[i], khs[i], (((1,), (1,)), ((), ())),
            preferred_element_type=f32)).astype(bf16) for i in range(NH)]

        for i in range(NH):
            tscr[i] = ts[i]
            qkscr[i] = qks[i]
            qscr[:, i * PHI:(i + 1) * PHI] = qs[i]
            kscr[:, i * PHI:(i + 1) * PHI] = khs[i]
            kbscr[:, i * PHI:(i + 1) * PHI] = kbs[i]
            bvscr[:, i * DK:(i + 1) * DK] = bvs[i]

    @pl.when(c == 0)
    def _():
        wstate[...] = jnp.zeros_like(wstate)
        _prep(x_ref[...])

    woh = wo_ref[...].astype(bf16)
    wts = [wstate[i] for i in range(NH)]
    wths = [w.astype(bf16) for w in wts]
    b_rhss = [(bvscr[:, i * DK:(i + 1) * DK]
               - jnp.dot(kbscr[:, i * PHI:(i + 1) * PHI], wths[i],
                         preferred_element_type=f32)).astype(bf16)
              for i in range(NH)]
    us = [jnp.dot(tscr[i], b_rhss[i], preferred_element_type=f32)
          for i in range(NH)]
    uhs = [u.astype(bf16) for u in us]
    o_hs = [jnp.dot(qscr[:, i * PHI:(i + 1) * PHI], wths[i],
                    preferred_element_type=f32)
            + jnp.dot(qkscr[i], uhs[i], preferred_element_type=f32)
            for i in range(NH)]
    for i in range(NH):
        wstate[i] = wts[i] + jax.lax.dot_general(
            kscr[:, i * PHI:(i + 1) * PHI], uhs[i],
            (((0,), (0,)), ((), ())), preferred_element_type=f32)

    for g in range(G):
        o_full = jnp.concatenate(
            o_hs[g * H:(g + 1) * H], axis=-1).astype(bf16)
        o_ref[:, g * D:(g + 1) * D] = (
            jnp.dot(o_full, woh, preferred_element_type=f32) + bo_ref[0, :])

    _prep(x2_ref[...])


def kernel(x, Wq, Wk, Wv, Wg, Wo, bo):
    S, B, D = x.shape
    H = Wg.shape[1]
    DK = Wq.shape[1] // H
    PHI = 2 * DK
    C = _C
    NC = S // C

    G = _G
    x2d = x.reshape(S, B * D)
    body = functools.partial(_fwa_body, H, DK, G)
    out = pl.pallas_call(
        body,
        grid=(B // G, NC),
        in_specs=[
            pl.BlockSpec((C, G * D), lambda b, c: (c, b)),
            pl.BlockSpec((C, G * D),
                         lambda b, c: (jnp.minimum(c + 1, NC - 1), b)),
            pl.BlockSpec((D, H * DK), lambda b, c: (0, 0)),
            pl.BlockSpec((D, H * DK), lambda b, c: (0, 0)),
            pl.BlockSpec((D, H * DK), lambda b, c: (0, 0)),
            pl.BlockSpec((D, H), lambda b, c: (0, 0)),
            pl.BlockSpec((D, D), lambda b, c: (0, 0)),
            pl.BlockSpec((1, D), lambda b, c: (0, 0)),
        ],
        out_specs=pl.BlockSpec((C, G * D), lambda b, c: (c, b)),
        out_shape=jax.ShapeDtypeStruct((S, B * D), x.dtype),
        scratch_shapes=[
            pltpu.VMEM((G * H, PHI, DK), jnp.float32),
            pltpu.VMEM((G * H, C, C), jnp.bfloat16),
            pltpu.VMEM((G * H, C, C), jnp.bfloat16),
            pltpu.VMEM((C, G * H * PHI), jnp.bfloat16),
            pltpu.VMEM((C, G * H * PHI), jnp.bfloat16),
            pltpu.VMEM((C, G * H * PHI), jnp.bfloat16),
            pltpu.VMEM((C, G * H * DK), jnp.bfloat16),
        ],
        compiler_params=pltpu.CompilerParams(
            dimension_semantics=("parallel", "arbitrary")),
    )(x2d, x2d, Wq, Wk, Wv, Wg, Wo, bo.reshape(1, D))
    return out.reshape(S, B, D)
